# baseline (speedup 1.0000x reference)
import jax
import jax.numpy as jnp
from jax import lax
from jax.experimental import pallas as pl
from jax.experimental.pallas import tpu as pltpu
from jax.experimental.pallas import tpu_sc as plsc

B = 16384
N_NUM = 13
N_CAT = 26
VOCAB = 100000
EMB_DIM = 32
EPS = 1e-08
N_COLS = N_NUM + N_CAT
OUT_D = N_NUM + N_CAT * EMB_DIM

NC = 2
NS = 16
NW = NC * NS
B_PER_W = B // NW
R = 8
N_CHUNKS = B_PER_W // R
N_PAIRS = N_CHUNKS // 2
LANES = 16
HALF = EMB_DIM // LANES
SUPER_D = 128
SUPER_ROWS = N_CAT * VOCAB * EMB_DIM // SUPER_D
LOOKUPS = N_CAT * R
N_STREAMS = 2
PER_STREAM = LOOKUPS // N_STREAMS


def _body(xt_hbm, tab_hbm, mean_hbm, std_hbm, out_hbm,
          xwbuf, idx_a, sub_a, idx_b, sub_b, cat_a, cat_b, obuf_a, obuf_b,
          mean_v, std_v, gsem_a, gsem_b, wsem_a, wsem_b):
  wid = lax.axis_index("s") * NC + lax.axis_index("c")

  pltpu.sync_copy(mean_hbm, mean_v)
  pltpu.sync_copy(std_hbm, std_v)
  pltpu.sync_copy(xt_hbm.at[:, pl.ds(wid * B_PER_W, B_PER_W)], xwbuf)

  iota = lax.iota(jnp.int32, LANES)
  lane_f = lax.shift_right_logical(iota, 3)
  lane_r = lax.bitwise_and(iota, 7)
  mask13 = iota < 13

  def idx_build(ch, idxs, subs):
    col = ch * R + lane_r
    for j in range(N_CAT // 2):
      fvec = lane_f + (2 * j)
      ids = plsc.load_gather(xwbuf, [fvec + N_NUM, col])
      gid = ids.astype(jnp.int32) + fvec * VOCAB
      idxs[pl.ds(2 * j * R, LANES)] = lax.shift_right_logical(gid, 2)
      subs[pl.ds(2 * j * R, LANES)] = lax.bitwise_and(gid, 3)

  def fire(idxs, cat, gsem):
    for g in range(N_STREAMS):
      sl = pl.ds(g * PER_STREAM, PER_STREAM)
      pltpu.async_copy(tab_hbm.at[idxs.at[sl]], cat.at[sl], gsem)

  def drain_gather(idxs, cat, gsem):
    for g in range(N_STREAMS):
      sl = pl.ds(g * PER_STREAM, PER_STREAM)
      pltpu.make_async_copy(tab_hbm.at[idxs.at[sl]], cat.at[sl], gsem).wait()

  def consume(ch, idxs, subs, cat, obuf, gsem, wsem, i, first_guard):
    @pl.when(first_guard)
    def _():
      pltpu.make_async_copy(obuf, out_hbm.at[pl.ds(0, R)], wsem).wait()

    drain_gather(idxs, cat, gsem)

    def row_body(r, carry):
      rv = jnp.full((LANES,), 0, jnp.int32) + r
      for f in range(N_CAT):
        kv = jnp.full((LANES,), f * R, jnp.int32) + r
        sub = plsc.load_gather(subs, [kv])
        src0 = sub * EMB_DIM + iota
        for h in range(HALF):
          v = plsc.load_gather(cat, [kv, src0 + (h * LANES)])
          dst_c = iota + (N_NUM + f * EMB_DIM + h * LANES)
          plsc.store_scatter(obuf, [rv, dst_c], v)
      return carry

    lax.fori_loop(0, R, row_body, 0)

    m = mean_v[...]
    s = std_v[...] + EPS
    def num_body(r, carry):
      v = plsc.load_gather(xwbuf, [iota, jnp.full((LANES,), 0, jnp.int32) + ch * R + r])
      plsc.store_scatter(obuf, [jnp.full((LANES,), 0, jnp.int32) + r, iota],
                         (v - m) / s, mask=mask13)
      return carry

    lax.fori_loop(0, R, num_body, 0)

    base = wid * B_PER_W + ch * R
    pltpu.async_copy(obuf, out_hbm.at[pl.ds(base, R)], wsem)

  def pair_body(i, carry):
    ch0 = 2 * i
    ch1 = 2 * i + 1

    idx_build(ch1, idx_b, sub_b)
    fire(idx_b, cat_b, gsem_b)

    consume(ch0, idx_a, sub_a, cat_a, obuf_a, gsem_a, wsem_a, i, i > 0)

    @pl.when(i < N_PAIRS - 1)
    def _():
      idx_build(ch0 + 2, idx_a, sub_a)
      fire(idx_a, cat_a, gsem_a)

    consume(ch1, idx_b, sub_b, cat_b, obuf_b, gsem_b, wsem_b, i, i > 0)
    return carry

  idx_build(0, idx_a, sub_a)
  fire(idx_a, cat_a, gsem_a)

  lax.fori_loop(0, N_PAIRS, pair_body, 0)

  pltpu.make_async_copy(obuf_a, out_hbm.at[pl.ds(0, R)], wsem_a).wait()
  pltpu.make_async_copy(obuf_b, out_hbm.at[pl.ds(0, R)], wsem_b).wait()


@jax.jit
def _run(xt, tab_flat, mean16, std16):
  mesh = plsc.VectorSubcoreMesh(core_axis_name="c", subcore_axis_name="s",
                                num_cores=NC, num_subcores=NS)
  return pl.kernel(
      _body,
      out_type=jax.ShapeDtypeStruct((B, OUT_D), jnp.float32),
      mesh=mesh,
      compiler_params=pltpu.CompilerParams(needs_layout_passes=False),
      scratch_types=[
          pltpu.VMEM((N_COLS, B_PER_W), jnp.float32),
          pltpu.VMEM((LOOKUPS,), jnp.int32),
          pltpu.VMEM((LOOKUPS,), jnp.int32),
          pltpu.VMEM((LOOKUPS,), jnp.int32),
          pltpu.VMEM((LOOKUPS,), jnp.int32),
          pltpu.VMEM((LOOKUPS, SUPER_D), jnp.float32),
          pltpu.VMEM((LOOKUPS, SUPER_D), jnp.float32),
          pltpu.VMEM((R, OUT_D), jnp.float32),
          pltpu.VMEM((R, OUT_D), jnp.float32),
          pltpu.VMEM((LANES,), jnp.float32),
          pltpu.VMEM((LANES,), jnp.float32),
          pltpu.SemaphoreType.DMA,
          pltpu.SemaphoreType.DMA,
          pltpu.SemaphoreType.DMA,
          pltpu.SemaphoreType.DMA,
      ],
  )(xt, tab_flat, mean16, std16)


def kernel(x, tables, mean, std):
  tab_flat = tables.reshape(SUPER_ROWS, SUPER_D)
  mean16 = jnp.zeros((LANES,), jnp.float32).at[:N_NUM].set(mean)
  std16 = jnp.ones((LANES,), jnp.float32).at[:N_NUM].set(std)
  return _run(x.T, tab_flat, mean16, std16)
